# baseline (speedup 1.0000x reference)
import functools

import jax
import jax.numpy as jnp
from jax import lax
from jax.experimental import pallas as pl
from jax.experimental.pallas import tpu as pltpu
from jax.experimental.pallas import tpu_sc as plsc

_NC = 2
_NS = 16
_NW = _NC * _NS


def _sc_gather(user_emb, item_emb, user_id, item_id):
    B = user_id.shape[0]
    D = user_emb.shape[1]
    b_per_w = B // _NW
    mesh = plsc.VectorSubcoreMesh(core_axis_name="c", subcore_axis_name="s")

    @functools.partial(
        pl.kernel,
        mesh=mesh,
        compiler_params=pltpu.CompilerParams(use_tc_tiling_on_sc=False),
        out_type=(
            jax.ShapeDtypeStruct((B, D), jnp.float32),
            jax.ShapeDtypeStruct((B, D), jnp.float32),
        ),
        scratch_types=[
            pltpu.VMEM((b_per_w,), jnp.int32),
            pltpu.VMEM((b_per_w,), jnp.int32),
            pltpu.VMEM((b_per_w, D), jnp.float32),
            pltpu.VMEM((b_per_w, D), jnp.float32),
            pltpu.SemaphoreType.DMA,
            pltpu.SemaphoreType.DMA,
        ],
    )
    def gather_kernel(uemb_hbm, iemb_hbm, uid_hbm, iid_hbm, u_out, v_out,
                      uidx_v, iidx_v, urows_v, irows_v, sem_u, sem_i):
        wid = lax.axis_index("s") * _NC + lax.axis_index("c")
        base = wid * b_per_w
        pltpu.sync_copy(uid_hbm.at[pl.ds(base, b_per_w)], uidx_v)
        pltpu.sync_copy(iid_hbm.at[pl.ds(base, b_per_w)], iidx_v)
        cu = pltpu.async_copy(uemb_hbm.at[uidx_v], urows_v, sem_u)
        ci = pltpu.async_copy(iemb_hbm.at[iidx_v], irows_v, sem_i)
        cu.wait()
        ci.wait()
        pltpu.sync_copy(urows_v, u_out.at[pl.ds(base, b_per_w)])
        pltpu.sync_copy(irows_v, v_out.at[pl.ds(base, b_per_w)])

    return gather_kernel(user_emb, item_emb, user_id, item_id)


def _tc_head(u, v, W1u, W1v, b1, W2, b2, W3, bias0):
    B = u.shape[0]
    D = u.shape[1]
    blk = 4096
    dn = (((1,), (1,)), ((), ()))

    def body(u_ref, v_ref, w1u_ref, w1v_ref, b1_ref, w2_ref, b2_ref, w3_ref,
             bias_ref, o_ref):
        uu = u_ref[...]
        vv = v_ref[...]
        fm = jnp.sum(uu + vv + uu * vv, axis=1)
        h1 = jnp.maximum(
            lax.dot_general(uu, w1u_ref[...], dn,
                            preferred_element_type=jnp.float32)
            + lax.dot_general(vv, w1v_ref[...], dn,
                              preferred_element_type=jnp.float32)
            + b1_ref[...], 0.0)
        h2 = jnp.maximum(
            lax.dot_general(h1, w2_ref[...], dn,
                            preferred_element_type=jnp.float32)
            + b2_ref[...], 0.0)
        deep = lax.dot_general(h2, w3_ref[...], dn,
                               preferred_element_type=jnp.float32)[:, 0]
        o_ref[...] = fm + deep + bias_ref[0, 0]

    H1 = W1u.shape[0]
    H2 = W2.shape[0]
    return pl.pallas_call(
        body,
        grid=(B // blk,),
        in_specs=[
            pl.BlockSpec((blk, D), lambda i: (i, 0)),
            pl.BlockSpec((blk, D), lambda i: (i, 0)),
            pl.BlockSpec((H1, D), lambda i: (0, 0)),
            pl.BlockSpec((H1, D), lambda i: (0, 0)),
            pl.BlockSpec((1, H1), lambda i: (0, 0)),
            pl.BlockSpec((H2, H1), lambda i: (0, 0)),
            pl.BlockSpec((1, H2), lambda i: (0, 0)),
            pl.BlockSpec((1, H2), lambda i: (0, 0)),
            pl.BlockSpec((1, 1), lambda i: (0, 0)),
        ],
        out_specs=pl.BlockSpec((blk,), lambda i: (i,)),
        out_shape=jax.ShapeDtypeStruct((B,), jnp.float32),
    )(u, v, W1u, W1v, b1, W2, b2, W3, bias0)


def kernel(user_id, item_id, user_emb, item_emb, fm_bias, W1, b1, W2, b2,
           W3, b3):
    D = user_emb.shape[1]
    u, v = _sc_gather(user_emb, item_emb,
                      user_id.astype(jnp.int32), item_id.astype(jnp.int32))
    W1u = W1[:, :D]
    W1v = W1[:, D:]
    bias0 = (fm_bias + b3).reshape(1, 1)
    return _tc_head(u, v, W1u, W1v, b1.reshape(1, -1), W2,
                    b2.reshape(1, -1), W3, bias0)
